# Initial kernel scaffold; baseline (speedup 1.0000x reference)
import jax
import jax.numpy as jnp
from jax.experimental import pallas as pl


def kernel(x, embed, fc_w, fc_b):
    raise NotImplementedError("write your pallas kernel here")


# speedup vs baseline: 12.5472x; 12.5472x over previous
import functools

import jax
import jax.numpy as jnp
from jax import lax
from jax.experimental import pallas as pl
from jax.experimental.pallas import tpu as pltpu
from jax.experimental.pallas import tpu_sc as plsc

B = 16384
H = 200
D = 32
C = 16
HALF = 100
NHALF = B * 2

NC = 2
NS = 16
NW = NC * NS

HW_PER_W = NHALF // NW
ROWS_PER_W = B // NW
CH_H = 64
NCHUNK = HW_PER_W // CH_H
G = 4


def _sc_body(x2_hbm, tab_hbm, out_hbm, idx_v, rows_v, out_v, sems):
  cid = lax.axis_index("c")
  sid = lax.axis_index("s")
  wid = sid * NC + cid
  hbase0 = wid * HW_PER_W
  obase = wid * ROWS_PER_W

  zero = jnp.zeros((16,), jnp.float32)

  def chunk_body(c, _):
    hb0 = hbase0 + c * CH_H
    pltpu.sync_copy(x2_hbm.at[pl.ds(hb0, CH_H), :], idx_v)

    def group_body(i, _):
      gb = i * G
      cps = []
      for g in range(G):
        cps.append(
            pltpu.async_copy(tab_hbm.at[idx_v.at[gb + g]], rows_v.at[g],
                             sems.at[g]))
      for g in range(G):
        cps[g].wait()

        def red_body(j, carry):
          a0, a1 = carry
          for k in range(4):
            jj = j * 4 + k
            a0 = a0 + rows_v[g, jj, 0:16]
            a1 = a1 + rows_v[g, jj, 16:32]
          return a0, a1

        a0, a1 = lax.fori_loop(0, HALF // 4, red_body, (zero, zero))
        slot = (c * CH_H + gb) // 2 + g // 2
        if g % 2 == 0:
          out_v[slot, 0:16] = a0
          out_v[slot, 16:32] = a1
        else:
          out_v[slot, 0:16] = out_v[slot, 0:16] + a0
          out_v[slot, 16:32] = out_v[slot, 16:32] + a1
      return _

    lax.fori_loop(0, CH_H // G, group_body, None)
    return _

  lax.fori_loop(0, NCHUNK, chunk_body, None)
  pltpu.sync_copy(out_v, out_hbm.at[pl.ds(obase, ROWS_PER_W), :])


@jax.jit
def _sc_pool(x2, embed):
  mesh = plsc.VectorSubcoreMesh(
      core_axis_name="c", subcore_axis_name="s", num_cores=NC,
      num_subcores=NS)
  f = pl.kernel(
      _sc_body,
      out_type=jax.ShapeDtypeStruct((B, D), jnp.float32),
      mesh=mesh,
      scratch_types=[
          pltpu.VMEM((CH_H, HALF), jnp.int32),
          pltpu.VMEM((G, HALF, D), jnp.float32),
          pltpu.VMEM((ROWS_PER_W, D), jnp.float32),
          pltpu.SemaphoreType.DMA((G,)),
      ],
      compiler_params=pltpu.CompilerParams(use_tc_tiling_on_sc=False),
  )
  return f(x2, embed)


def _tc_body(ms_ref, wt_ref, b_ref, out_ref):
  m = ms_ref[...] * jnp.float32(1.0 / H)
  logits = jnp.dot(m, wt_ref[...], preferred_element_type=jnp.float32)
  logits = logits + b_ref[...]
  mx = jnp.max(logits, axis=1, keepdims=True)
  s = logits - mx
  lse = jnp.log(jnp.sum(jnp.exp(s), axis=1, keepdims=True))
  out_ref[...] = s - lse


@jax.jit
def _tc_head(msum, wt, b2):
  blk = 2048
  return pl.pallas_call(
      _tc_body,
      grid=(B // blk,),
      in_specs=[
          pl.BlockSpec((blk, D), lambda i: (i, 0)),
          pl.BlockSpec((D, C), lambda i: (0, 0)),
          pl.BlockSpec((1, C), lambda i: (0, 0)),
      ],
      out_specs=pl.BlockSpec((blk, C), lambda i: (i, 0)),
      out_shape=jax.ShapeDtypeStruct((B, C), jnp.float32),
  )(msum, wt, b2)


def kernel(x, embed, fc_w, fc_b):
  x2 = x.astype(jnp.int32).reshape(NHALF, HALF)
  msum = _sc_pool(x2, embed)
  return _tc_head(msum, fc_w.T, fc_b.reshape(1, C))


# speedup vs baseline: 16.3821x; 1.3056x over previous
import functools

import jax
import jax.numpy as jnp
from jax import lax
from jax.experimental import pallas as pl
from jax.experimental.pallas import tpu as pltpu
from jax.experimental.pallas import tpu_sc as plsc

B = 16384
H = 200
D = 32
C = 16
HALF = 100
NHALF = B * 2

NC = 2
NS = 16
NW = NC * NS

HW_PER_W = NHALF // NW
ROWS_PER_W = B // NW
CH_H = 128
NCHUNK = HW_PER_W // CH_H
G = 8


def _sc_body(x2_hbm, tab_hbm, out_hbm, idx_v, rows_v, out_v, isems, gsems):
  cid = lax.axis_index("c")
  sid = lax.axis_index("s")
  wid = sid * NC + cid
  hbase0 = wid * HW_PER_W
  obase = wid * ROWS_PER_W

  zero = jnp.zeros((16,), jnp.float32)

  def fire_idx(c, buf):
    pltpu.async_copy(x2_hbm.at[pl.ds(hbase0 + c * CH_H, CH_H), :],
                     idx_v.at[buf], isems.at[buf])

  def wait_idx(buf):
    pltpu.make_async_copy(x2_hbm.at[pl.ds(hbase0, CH_H), :], idx_v.at[buf],
                          isems.at[buf]).wait()

  def fire_gather(p, h, g):
    pltpu.async_copy(tab_hbm.at[idx_v.at[p, h]], rows_v.at[g], gsems.at[g])

  def wait_gather(g):
    pltpu.make_async_copy(tab_hbm.at[pl.ds(0, HALF), :], rows_v.at[g],
                          gsems.at[g]).wait()

  def reduce_store(c, hb, g):
    def red_body(j, acc):
      acc = list(acc)
      for k in range(4):
        jj = j * 4 + k
        acc[2 * k] = acc[2 * k] + rows_v[g, jj, 0:16]
        acc[2 * k + 1] = acc[2 * k + 1] + rows_v[g, jj, 16:32]
      return tuple(acc)

    acc = lax.fori_loop(0, HALF // 4, red_body, (zero,) * 8)
    b0 = (acc[0] + acc[2]) + (acc[4] + acc[6])
    b1 = (acc[1] + acc[3]) + (acc[5] + acc[7])
    slot = c * (CH_H // 2) + hb // 2 + g // 2
    if g % 2 == 0:
      out_v[slot, 0:16] = b0
      out_v[slot, 16:32] = b1
    else:
      out_v[slot, 0:16] = out_v[slot, 0:16] + b0
      out_v[slot, 16:32] = out_v[slot, 16:32] + b1

  fire_idx(0, 0)

  def pair_body(pair, _):
    for p in range(2):
      c = pair * 2 + p
      wait_idx(p)

      @pl.when(c + 1 < NCHUNK)
      def _prefetch():
        fire_idx(c + 1, 1 - p)

      for g in range(G):
        fire_gather(p, g, g)

      def ring_body(i, _):
        hb = i * G
        for g in range(G):
          wait_gather(g)
          reduce_store(c, hb, g)
          fire_gather(p, hb + G + g, g)
        return _

      lax.fori_loop(0, CH_H // G - 1, ring_body, None)
      for g in range(G):
        wait_gather(g)
        reduce_store(c, CH_H - G, g)
    return _

  lax.fori_loop(0, NCHUNK // 2, pair_body, None)
  pltpu.sync_copy(out_v, out_hbm.at[pl.ds(obase, ROWS_PER_W), :])


@jax.jit
def _sc_pool(x2, embed):
  mesh = plsc.VectorSubcoreMesh(
      core_axis_name="c", subcore_axis_name="s", num_cores=NC,
      num_subcores=NS)
  f = pl.kernel(
      _sc_body,
      out_type=jax.ShapeDtypeStruct((B, D), jnp.float32),
      mesh=mesh,
      scratch_types=[
          pltpu.VMEM((2, CH_H, HALF), jnp.int32),
          pltpu.VMEM((G, HALF, D), jnp.float32),
          pltpu.VMEM((ROWS_PER_W, D), jnp.float32),
          pltpu.SemaphoreType.DMA((2,)),
          pltpu.SemaphoreType.DMA((G,)),
      ],
      compiler_params=pltpu.CompilerParams(use_tc_tiling_on_sc=False),
  )
  return f(x2, embed)


def _tc_body(ms_ref, wt_ref, b_ref, out_ref):
  m = ms_ref[...] * jnp.float32(1.0 / H)
  logits = jnp.dot(m, wt_ref[...], preferred_element_type=jnp.float32)
  logits = logits + b_ref[...]
  mx = jnp.max(logits, axis=1, keepdims=True)
  s = logits - mx
  lse = jnp.log(jnp.sum(jnp.exp(s), axis=1, keepdims=True))
  out_ref[...] = s - lse


@jax.jit
def _tc_head(msum, wt, b2):
  blk = 2048
  return pl.pallas_call(
      _tc_body,
      grid=(B // blk,),
      in_specs=[
          pl.BlockSpec((blk, D), lambda i: (i, 0)),
          pl.BlockSpec((D, C), lambda i: (0, 0)),
          pl.BlockSpec((1, C), lambda i: (0, 0)),
      ],
      out_specs=pl.BlockSpec((blk, C), lambda i: (i, 0)),
      out_shape=jax.ShapeDtypeStruct((B, C), jnp.float32),
  )(msum, wt, b2)


def kernel(x, embed, fc_w, fc_b):
  x2 = x.astype(jnp.int32).reshape(NHALF, HALF)
  msum = _sc_pool(x2, embed)
  return _tc_head(msum, fc_w.T, fc_b.reshape(1, C))
